# baseline (speedup 1.0000x reference)
import functools
import jax
import jax.numpy as jnp
from jax import lax
from jax.experimental import pallas as pl
from jax.experimental.pallas import tpu as pltpu
from jax.experimental.pallas import tpu_sc as plsc

H = 16
CHUNK = 64
NB = 64
SEQ = 2048
DH = 64
NC = SEQ // CHUNK
BN = 256


def _ln(x, g, b):
    mu = jnp.mean(x, axis=-1, keepdims=True)
    var = jnp.mean((x - mu) * (x - mu), axis=-1, keepdims=True)
    return (x - mu) / jnp.sqrt(var + 1e-5) * g + b


def _k1_body(x_ref, g_ref, b_ref, wqk_ref, wv_ref, out_ref):
    x = x_ref[0]
    h1 = _ln(x, g_ref[0], b_ref[0])
    qk = jnp.dot(h1, wqk_ref[...], preferred_element_type=jnp.float32)
    v = jnp.dot(h1, wv_ref[...], preferred_element_type=jnp.float32)
    qkv = jnp.concatenate(
        [qk.reshape(BN, H, DH), v.reshape(BN, H, DH)], axis=-1)
    out_ref[0] = qkv


def _ln_qkv(x, g, b, wqk, wv):
    B = x.shape[0]
    return pl.pallas_call(
        _k1_body,
        grid=(B, SEQ // BN),
        in_specs=[
            pl.BlockSpec((1, BN, 1024), lambda i, r: (i, r, 0)),
            pl.BlockSpec((1, 1024), lambda i, r: (0, 0)),
            pl.BlockSpec((1, 1024), lambda i, r: (0, 0)),
            pl.BlockSpec((1024, 1024), lambda i, r: (0, 0)),
            pl.BlockSpec((1024, 1024), lambda i, r: (0, 0)),
        ],
        out_specs=pl.BlockSpec((1, BN, H, 128), lambda i, r: (i, r, 0, 0)),
        out_shape=jax.ShapeDtypeStruct((B, SEQ, H, 128), jnp.float32),
    )(x, g.reshape(1, 1024), b.reshape(1, 1024), wqk, wv)


def _k2_body(qkv_ref, rot_ref, idx_ref):
    bi = pl.program_id(0)
    x = qkv_ref[0]
    rot = rot_ref[...]
    iota = lax.broadcasted_iota(jnp.int32, (SEQ, NB), 1)
    jj = lax.broadcasted_iota(jnp.int32, (NB, NB), 0)
    kk = lax.broadcasted_iota(jnp.int32, (NB, NB), 1)
    tri = (jj < kk).astype(jnp.float32)
    cols = []
    for h in range(H):
        qk = x[:, h, 0:DH]
        pr = jnp.dot(qk, rot, preferred_element_type=jnp.float32)
        pc = jnp.concatenate([pr, -pr], axis=1)
        m = jnp.max(pc, axis=1, keepdims=True)
        bucket = jnp.min(jnp.where(pc == m, iota, NB), axis=1, keepdims=True)
        oh = (bucket == iota).astype(jnp.float32)
        cum = oh
        s = 1
        while s < SEQ:
            shifted = jnp.concatenate(
                [jnp.zeros((s, NB), jnp.float32), cum[:SEQ - s]], axis=0)
            cum = cum + shifted
            s *= 2
        totals = cum[SEQ - 1:SEQ, :]
        offs = jnp.dot(totals, tri, preferred_element_type=jnp.float32)
        rank = jnp.sum(oh * (cum - 1.0 + offs), axis=1, keepdims=True)
        cols.append(rank.astype(jnp.int32) + (bi * H + h) * SEQ)
    idx_ref[0] = jnp.concatenate(cols, axis=1)


def _lsh_idx(qkv, rot):
    B = qkv.shape[0]
    return pl.pallas_call(
        _k2_body,
        grid=(B,),
        in_specs=[
            pl.BlockSpec((1, SEQ, H, 128), lambda i: (i, 0, 0, 0)),
            pl.BlockSpec((64, 32), lambda i: (0, 0)),
        ],
        out_specs=pl.BlockSpec((1, SEQ, H), lambda i: (i, 0, 0)),
        out_shape=jax.ShapeDtypeStruct((B, SEQ, H), jnp.int32),
    )(qkv, rot)


def _k3_body(s_ref, o_ref):
    x = s_ref[0]
    qf = x[:, 0:DH]
    k = qf / (jnp.sqrt(jnp.sum(qf * qf, axis=1, keepdims=True)) + 1e-6)
    kc = k.reshape(NC, CHUNK, DH)
    vc = x[:, DH:].reshape(NC, CHUNK, DH)
    k2 = jnp.concatenate(
        [jnp.concatenate([kc[NC - 1:], kc[:NC - 1]], axis=0), kc], axis=1)
    v2 = jnp.concatenate(
        [jnp.concatenate([vc[NC - 1:], vc[:NC - 1]], axis=0), vc], axis=1)
    qc = qf.reshape(NC, CHUNK, DH)
    dots = lax.dot_general(
        qc, k2, (((2,), (2,)), ((0,), (0,))),
        preferred_element_type=jnp.float32) * (DH ** -0.5)
    ii = lax.broadcasted_iota(jnp.int32, (CHUNK, 2 * CHUNK), 0)
    jj = lax.broadcasted_iota(jnp.int32, (CHUNK, 2 * CHUNK), 1)
    mask = jj == ii + CHUNK
    dots = jnp.where(mask[None], -1e5, dots)
    m = jnp.max(dots, axis=2, keepdims=True)
    e = jnp.exp(dots - m)
    attn = e / jnp.sum(e, axis=2, keepdims=True)
    bo = lax.dot_general(
        attn, v2, (((2,), (1,)), ((0,), (0,))),
        preferred_element_type=jnp.float32)
    o_ref[0] = jnp.concatenate(
        [bo.reshape(SEQ, DH), jnp.zeros((SEQ, DH), jnp.float32)], axis=1)


def _attend(sqkv):
    BH = sqkv.shape[0]
    return pl.pallas_call(
        _k3_body,
        grid=(BH,),
        in_specs=[pl.BlockSpec((1, SEQ, 128), lambda i: (i, 0, 0))],
        out_specs=pl.BlockSpec((1, SEQ, 128), lambda i: (i, 0, 0)),
        out_shape=jax.ShapeDtypeStruct((BH, SEQ, 128), jnp.float32),
    )(sqkv)


FB = 2048


def _k45_body(x_ref, o_ref, wo_ref, g_ref, b_ref, w1_ref, b1_ref, w2_ref,
              b2_ref, out_ref, xn_s, h2_s):
    f = pl.program_id(2)

    @pl.when(f == 0)
    def _():
        o = o_ref[0][:, :, 0:DH].reshape(BN, 1024)
        xn = x_ref[0] + jnp.dot(o, wo_ref[...],
                                preferred_element_type=jnp.float32)
        xn_s[...] = xn
        h2_s[...] = _ln(xn, g_ref[0], b_ref[0])

    g1 = jax.nn.gelu(
        jnp.dot(h2_s[...], w1_ref[...], preferred_element_type=jnp.float32)
        + b1_ref[0])
    contrib = jnp.dot(g1, w2_ref[...], preferred_element_type=jnp.float32)

    @pl.when(f == 0)
    def _():
        out_ref[0] = xn_s[...] + b2_ref[0] + contrib

    @pl.when(f > 0)
    def _():
        out_ref[0] = out_ref[0] + contrib


def _block_tail(x, o, wo, g, b, w1, b1, w2, b2):
    B = x.shape[0]
    return pl.pallas_call(
        _k45_body,
        grid=(B, SEQ // BN, 4096 // FB),
        in_specs=[
            pl.BlockSpec((1, BN, 1024), lambda i, r, f: (i, r, 0)),
            pl.BlockSpec((1, BN, H, 128), lambda i, r, f: (i, r, 0, 0)),
            pl.BlockSpec((1024, 1024), lambda i, r, f: (0, 0)),
            pl.BlockSpec((1, 1024), lambda i, r, f: (0, 0)),
            pl.BlockSpec((1, 1024), lambda i, r, f: (0, 0)),
            pl.BlockSpec((1024, FB), lambda i, r, f: (0, f)),
            pl.BlockSpec((1, FB), lambda i, r, f: (0, f)),
            pl.BlockSpec((FB, 1024), lambda i, r, f: (f, 0)),
            pl.BlockSpec((1, 1024), lambda i, r, f: (0, 0)),
        ],
        out_specs=pl.BlockSpec((1, BN, 1024), lambda i, r, f: (i, r, 0)),
        out_shape=jax.ShapeDtypeStruct((B, SEQ, 1024), jnp.float32),
        scratch_shapes=[
            pltpu.VMEM((BN, 1024), jnp.float32),
            pltpu.VMEM((BN, 1024), jnp.float32),
        ],
    )(x, o, wo, g.reshape(1, 1024), b.reshape(1, 1024), w1,
      b1.reshape(1, 4096), w2, b2.reshape(1, 1024))


def _k6_body(x_ref, w_ref, o_ref):
    xm = jnp.mean(x_ref[0], axis=0, keepdims=True)
    o_ref[0] = jnp.dot(xm, w_ref[...], preferred_element_type=jnp.float32)


def _mean_fc(x, fc_w):
    B = x.shape[0]
    out = pl.pallas_call(
        _k6_body,
        grid=(B,),
        in_specs=[
            pl.BlockSpec((1, SEQ, 1024), lambda i: (i, 0, 0)),
            pl.BlockSpec((1024, 1024), lambda i: (0, 0)),
        ],
        out_specs=pl.BlockSpec((1, 1, 1024), lambda i: (i, 0, 0)),
        out_shape=jax.ShapeDtypeStruct((B, 1, 1024), jnp.float32),
    )(x, fc_w)
    return out.reshape(B, 1024)


_NSLOT = 4
_PIPE = True


def _sc_gather_serial(table, idx2d, chunk):
    info = plsc.get_sparse_core_info()
    nw = info.num_cores * info.num_subcores
    R = idx2d.shape[0] * idx2d.shape[1]
    D = table.shape[1]
    rpw = R // nw
    nch = rpw // chunk
    idx = idx2d.reshape(-1)
    mesh = plsc.VectorSubcoreMesh(core_axis_name="c", subcore_axis_name="s")

    @functools.partial(
        pl.kernel, mesh=mesh,
        out_type=jax.ShapeDtypeStruct((R, D), jnp.float32),
        scratch_types=[
            pltpu.VMEM((chunk,), jnp.int32),
            pltpu.VMEM((chunk, D), jnp.float32),
            pltpu.SemaphoreType.DMA,
        ],
    )
    def k(table_hbm, idx_hbm, out_hbm, idx_v, buf, sem):
        wid = lax.axis_index("s") * info.num_cores + lax.axis_index("c")
        base = wid * rpw

        def body(j, carry):
            off = base + j * chunk
            pltpu.sync_copy(idx_hbm.at[pl.ds(off, chunk)], idx_v)
            pltpu.async_copy(table_hbm.at[idx_v], buf, sem).wait()
            pltpu.sync_copy(buf, out_hbm.at[pl.ds(off, chunk)])
            return carry

        lax.fori_loop(0, nch, body, 0)

    return k(table, idx)


def _sc_scatter_serial(src, idx2d, chunk):
    info = plsc.get_sparse_core_info()
    nw = info.num_cores * info.num_subcores
    R = src.shape[0]
    D = src.shape[1]
    rpw = R // nw
    nch = rpw // chunk
    mesh = plsc.VectorSubcoreMesh(core_axis_name="c", subcore_axis_name="s")

    @functools.partial(
        pl.kernel, mesh=mesh,
        out_type=jax.ShapeDtypeStruct((R, D), jnp.float32),
        scratch_types=[
            pltpu.VMEM((nch, chunk), jnp.int32),
            pltpu.VMEM((chunk, D), jnp.float32),
            pltpu.SemaphoreType.DMA,
        ],
    )
    def k(src_hbm, idx_hbm, out_hbm, idx_v, buf, sem):
        wid = lax.axis_index("s") * info.num_cores + lax.axis_index("c")
        base = wid * rpw
        pltpu.sync_copy(idx_hbm.at[pl.ds(wid * nch, nch)], idx_v)

        def body(j, carry):
            pltpu.sync_copy(src_hbm.at[pl.ds(base + j * chunk, chunk)], buf)
            pltpu.async_copy(buf, out_hbm.at[idx_v.at[j]], sem).wait()
            return carry

        lax.fori_loop(0, nch, body, 0)

    return k(src, idx2d)


def _sc_gather(table, idx2d, chunk):
    info = plsc.get_sparse_core_info()
    nw = info.num_cores * info.num_subcores
    R = idx2d.shape[0] * idx2d.shape[1]
    D = table.shape[1]
    rpw = R // nw
    nch = rpw // chunk
    mesh = plsc.VectorSubcoreMesh(core_axis_name="c", subcore_axis_name="s")

    @functools.partial(
        pl.kernel, mesh=mesh,
        out_type=jax.ShapeDtypeStruct((R, D), jnp.float32),
        scratch_types=[
            pltpu.VMEM((nch, chunk), jnp.int32),
            [pltpu.VMEM((chunk,), jnp.int32) for _ in range(_NSLOT)],
            [pltpu.VMEM((chunk, D), jnp.float32) for _ in range(_NSLOT)],
            [pltpu.SemaphoreType.DMA for _ in range(_NSLOT)],
        ],
    )
    def k(table_hbm, idx_hbm, out_hbm, idx_v, idx1s, bufs, sems):
        wid = lax.axis_index("s") * info.num_cores + lax.axis_index("c")
        base = wid * rpw
        pltpu.sync_copy(idx_hbm.at[pl.ds(wid * nch, nch)], idx_v)

        def start(j, s):
            for i in range(chunk // 16):
                idx1s[s][pl.ds(i * 16, 16)] = idx_v[j, pl.ds(i * 16, 16)]
            pltpu.async_copy(table_hbm.at[idx1s[s]], bufs[s], sems[s])

        for p in range(_NSLOT - 1):
            start(p, p)

        def body(j4, carry):
            for p in range(_NSLOT):
                j = j4 * _NSLOT + p
                pltpu.make_async_copy(
                    table_hbm.at[idx1s[p]], bufs[p], sems[p]).wait()
                pltpu.sync_copy(bufs[p], out_hbm.at[pl.ds(base + j * chunk,
                                                          chunk)])
                nxt = j + _NSLOT - 1
                s = (p + _NSLOT - 1) % _NSLOT

                @pl.when(nxt < nch)
                def _():
                    start(nxt, s)
            return carry

        lax.fori_loop(0, nch // _NSLOT, body, 0)

    return k(table, idx2d)


def _sc_scatter(src, idx2d, chunk):
    info = plsc.get_sparse_core_info()
    nw = info.num_cores * info.num_subcores
    R = src.shape[0]
    D = src.shape[1]
    rpw = R // nw
    nch = rpw // chunk
    mesh = plsc.VectorSubcoreMesh(core_axis_name="c", subcore_axis_name="s")

    @functools.partial(
        pl.kernel, mesh=mesh,
        out_type=jax.ShapeDtypeStruct((R, D), jnp.float32),
        scratch_types=[
            pltpu.VMEM((nch, chunk), jnp.int32),
            [pltpu.VMEM((chunk, D), jnp.float32) for _ in range(_NSLOT)],
            [pltpu.SemaphoreType.DMA for _ in range(_NSLOT)],
        ],
    )
    def k(src_hbm, idx_hbm, out_hbm, idx_v, bufs, sems):
        wid = lax.axis_index("s") * info.num_cores + lax.axis_index("c")
        base = wid * rpw
        pltpu.sync_copy(idx_hbm.at[pl.ds(wid * nch, nch)], idx_v)
        for p in range(_NSLOT - 1):
            pltpu.sync_copy(src_hbm.at[pl.ds(base + p * chunk, chunk)],
                            bufs[p])
            pltpu.async_copy(bufs[p], out_hbm.at[idx_v.at[p]], sems[p])

        def body(j4, carry):
            for p in range(_NSLOT):
                j = j4 * _NSLOT + p
                pltpu.make_async_copy(
                    bufs[p], out_hbm.at[idx_v.at[j]], sems[p]).wait()
                nxt = j + _NSLOT - 1
                s = (p + _NSLOT - 1) % _NSLOT

                @pl.when(nxt < nch)
                def _():
                    pltpu.sync_copy(
                        src_hbm.at[pl.ds(base + nxt * chunk, chunk)], bufs[s])
                    pltpu.async_copy(bufs[s], out_hbm.at[idx_v.at[nxt]],
                                     sems[s])
            return carry

        lax.fori_loop(0, nch // _NSLOT, body, 0)

    return k(src, idx2d)


def _embed(tok_emb, ids):
    return _sc_gather_serial(tok_emb, ids.reshape(-1, 64), 64)


def _sort_rows(qkv_flat, idx):
    f = _sc_scatter if _PIPE else _sc_scatter_serial
    return f(qkv_flat, idx.reshape(-1, 128), 128)


def _unsort_rows(so_flat, idx):
    f = _sc_gather if _PIPE else _sc_gather_serial
    return f(so_flat, idx.reshape(-1, 128), 128)


def kernel(input_ids, tok_emb, pos_emb, Wqk, Wv, Wo, ln1_g, ln1_b, W1, b1,
           W2, b2, ln2_g, ln2_b, fc_W):
    B = input_ids.shape[0]
    slen = input_ids.shape[1]
    if slen < SEQ:
        pad = jnp.zeros((B, SEQ - slen), dtype=input_ids.dtype)
        input_ids = jnp.concatenate([input_ids, pad], axis=1)
    else:
        input_ids = input_ids[:, :SEQ]
    L = Wqk.shape[0]

    ids = input_ids.reshape(-1).astype(jnp.int32)
    x = _embed(tok_emb, ids).reshape(B, SEQ, 1024) + pos_emb[None]

    rkey = jax.random.key(42)
    rots = [
        jax.random.normal(jax.random.fold_in(rkey, i), (DH, NB // 2),
                          dtype=jnp.float32)
        for i in range(L)
    ]

    for i in range(L):
        qkv = _ln_qkv(x, ln1_g[i], ln1_b[i], Wqk[i], Wv[i])
        idx = _lsh_idx(qkv, rots[i]).reshape(-1)
        sqkv = _sort_rows(qkv.reshape(B * SEQ * H, 128), idx)
        so = _attend(sqkv.reshape(B * H, SEQ, 128))
        o = _unsort_rows(so.reshape(B * H * SEQ, 128), idx)
        x = _block_tail(x, o.reshape(B, SEQ, H, 128), Wo[i], ln2_g[i],
                        ln2_b[i], W1[i], b1[i], W2[i], b2[i])

    return _mean_fc(x, fc_W)


# speedup vs baseline: 1.0837x; 1.0837x over previous
import functools
import jax
import jax.numpy as jnp
from jax import lax
from jax.experimental import pallas as pl
from jax.experimental.pallas import tpu as pltpu
from jax.experimental.pallas import tpu_sc as plsc

H = 16
CHUNK = 64
NB = 64
SEQ = 2048
DH = 64
NC = SEQ // CHUNK
BN = 256


def _ln(x, g, b):
    mu = jnp.mean(x, axis=-1, keepdims=True)
    var = jnp.mean((x - mu) * (x - mu), axis=-1, keepdims=True)
    return (x - mu) / jnp.sqrt(var + 1e-5) * g + b


def _k1_body(x_ref, g_ref, b_ref, wqk_ref, wv_ref, out_ref):
    x = x_ref[0]
    h1 = _ln(x, g_ref[0], b_ref[0])
    qk = jnp.dot(h1, wqk_ref[...], preferred_element_type=jnp.float32)
    v = jnp.dot(h1, wv_ref[...], preferred_element_type=jnp.float32)
    qkv = jnp.concatenate(
        [qk.reshape(BN, H, DH), v.reshape(BN, H, DH)], axis=-1)
    out_ref[0] = qkv


def _ln_qkv(x, g, b, wqk, wv):
    B = x.shape[0]
    return pl.pallas_call(
        _k1_body,
        grid=(B, SEQ // BN),
        in_specs=[
            pl.BlockSpec((1, BN, 1024), lambda i, r: (i, r, 0)),
            pl.BlockSpec((1, 1024), lambda i, r: (0, 0)),
            pl.BlockSpec((1, 1024), lambda i, r: (0, 0)),
            pl.BlockSpec((1024, 1024), lambda i, r: (0, 0)),
            pl.BlockSpec((1024, 1024), lambda i, r: (0, 0)),
        ],
        out_specs=pl.BlockSpec((1, BN, H, 128), lambda i, r: (i, r, 0, 0)),
        out_shape=jax.ShapeDtypeStruct((B, SEQ, H, 128), jnp.float32),
    )(x, g.reshape(1, 1024), b.reshape(1, 1024), wqk, wv)


def _k2_body(qkv_ref, rot_ref, idx_ref):
    bi = pl.program_id(0)
    x = qkv_ref[0]
    rot = rot_ref[...]
    iota = lax.broadcasted_iota(jnp.int32, (SEQ, NB), 1)
    buckets = []
    for h in range(H):
        qk = x[:, h, 0:DH]
        pr = jnp.dot(qk, rot, preferred_element_type=jnp.float32)
        pc = jnp.concatenate([pr, -pr], axis=1)
        m = jnp.max(pc, axis=1, keepdims=True)
        buckets.append(
            jnp.min(jnp.where(pc == m, iota, NB), axis=1, keepdims=True))
    ball = jnp.concatenate(buckets, axis=1)
    i3 = lax.broadcasted_iota(jnp.int32, (SEQ, H, NB), 2)
    oh = (ball[:, :, None] == i3).astype(jnp.float32).reshape(SEQ, H * NB)
    blocks = oh.reshape(NC, CHUNK, H * NB)
    r1 = lax.broadcasted_iota(jnp.int32, (CHUNK, CHUNK), 0)
    r2 = lax.broadcasted_iota(jnp.int32, (CHUNK, CHUNK), 1)
    tri = jnp.broadcast_to((r2 <= r1).astype(jnp.float32)[None],
                           (NC, CHUNK, CHUNK))
    local = lax.dot_general(tri, blocks, (((2,), (1,)), ((0,), (0,))),
                            preferred_element_type=jnp.float32)
    bsum = jnp.sum(blocks, axis=1)
    inc = bsum
    s = 1
    while s < NC:
        inc = inc + jnp.concatenate(
            [jnp.zeros((s, H * NB), jnp.float32), inc[:NC - s]], axis=0)
        s *= 2
    pre = inc - bsum
    cum = (local + pre[:, None, :]).reshape(SEQ, H * NB)
    total = jnp.sum(oh, axis=0, keepdims=True)
    col = lax.broadcasted_iota(jnp.int32, (1, H * NB), 1)
    inc2 = total
    s = 1
    while s < NB:
        sh = jnp.concatenate(
            [jnp.zeros((1, s), jnp.float32), inc2[:, :H * NB - s]], axis=1)
        inc2 = inc2 + jnp.where(col % NB >= s, sh, 0.0)
        s *= 2
    offs = inc2 - total
    z = jnp.sum((oh * (cum - 1.0 + offs)).reshape(SEQ, H, NB), axis=2)
    base = (lax.broadcasted_iota(jnp.int32, (1, H), 1) + bi * H) * SEQ
    idx_ref[0] = z.astype(jnp.int32) + base


def _lsh_idx(qkv, rot):
    B = qkv.shape[0]
    return pl.pallas_call(
        _k2_body,
        grid=(B,),
        in_specs=[
            pl.BlockSpec((1, SEQ, H, 128), lambda i: (i, 0, 0, 0)),
            pl.BlockSpec((64, 32), lambda i: (0, 0)),
        ],
        out_specs=pl.BlockSpec((1, SEQ, H), lambda i: (i, 0, 0)),
        out_shape=jax.ShapeDtypeStruct((B, SEQ, H), jnp.int32),
    )(qkv, rot)


def _k3_body(s_ref, o_ref):
    x = s_ref[0]
    qf = x[:, 0:DH]
    k = qf / (jnp.sqrt(jnp.sum(qf * qf, axis=1, keepdims=True)) + 1e-6)
    kc = k.reshape(NC, CHUNK, DH)
    vc = x[:, DH:].reshape(NC, CHUNK, DH)
    k2 = jnp.concatenate(
        [jnp.concatenate([kc[NC - 1:], kc[:NC - 1]], axis=0), kc], axis=1)
    v2 = jnp.concatenate(
        [jnp.concatenate([vc[NC - 1:], vc[:NC - 1]], axis=0), vc], axis=1)
    qc = qf.reshape(NC, CHUNK, DH)
    dots = lax.dot_general(
        qc, k2, (((2,), (2,)), ((0,), (0,))),
        preferred_element_type=jnp.float32) * (DH ** -0.5)
    ii = lax.broadcasted_iota(jnp.int32, (CHUNK, 2 * CHUNK), 0)
    jj = lax.broadcasted_iota(jnp.int32, (CHUNK, 2 * CHUNK), 1)
    mask = jj == ii + CHUNK
    dots = jnp.where(mask[None], -1e5, dots)
    m = jnp.max(dots, axis=2, keepdims=True)
    e = jnp.exp(dots - m)
    attn = e / jnp.sum(e, axis=2, keepdims=True)
    bo = lax.dot_general(
        attn, v2, (((2,), (1,)), ((0,), (0,))),
        preferred_element_type=jnp.float32)
    o_ref[0] = jnp.concatenate(
        [bo.reshape(SEQ, DH), jnp.zeros((SEQ, DH), jnp.float32)], axis=1)


def _attend(sqkv):
    BH = sqkv.shape[0]
    return pl.pallas_call(
        _k3_body,
        grid=(BH,),
        in_specs=[pl.BlockSpec((1, SEQ, 128), lambda i: (i, 0, 0))],
        out_specs=pl.BlockSpec((1, SEQ, 128), lambda i: (i, 0, 0)),
        out_shape=jax.ShapeDtypeStruct((BH, SEQ, 128), jnp.float32),
    )(sqkv)


def _k4_body(x_ref, o_ref, wo_ref, g_ref, b_ref, xo_ref, h2_ref):
    o = o_ref[0][:, :, 0:DH].reshape(BN, 1024)
    xn = x_ref[0] + jnp.dot(o, wo_ref[...],
                            preferred_element_type=jnp.float32)
    xo_ref[0] = xn
    h2_ref[0] = _ln(xn, g_ref[0], b_ref[0])


def _oproj_ln(x, o, wo, g, b):
    B = x.shape[0]
    return pl.pallas_call(
        _k4_body,
        grid=(B, SEQ // BN),
        in_specs=[
            pl.BlockSpec((1, BN, 1024), lambda i, r: (i, r, 0)),
            pl.BlockSpec((1, BN, H, 128), lambda i, r: (i, r, 0, 0)),
            pl.BlockSpec((1024, 1024), lambda i, r: (0, 0)),
            pl.BlockSpec((1, 1024), lambda i, r: (0, 0)),
            pl.BlockSpec((1, 1024), lambda i, r: (0, 0)),
        ],
        out_specs=[
            pl.BlockSpec((1, BN, 1024), lambda i, r: (i, r, 0)),
            pl.BlockSpec((1, BN, 1024), lambda i, r: (i, r, 0)),
        ],
        out_shape=[
            jax.ShapeDtypeStruct((B, SEQ, 1024), jnp.float32),
            jax.ShapeDtypeStruct((B, SEQ, 1024), jnp.float32),
        ],
    )(x, o, wo, g.reshape(1, 1024), b.reshape(1, 1024))


def _k5a_body(h2_ref, w1_ref, b1_ref, g_ref):
    g_ref[0] = jax.nn.gelu(
        jnp.dot(h2_ref[0], w1_ref[...], preferred_element_type=jnp.float32)
        + b1_ref[0])


def _ffn_up(h2, w1, b1):
    B = h2.shape[0]
    return pl.pallas_call(
        _k5a_body,
        grid=(B, SEQ // BN),
        in_specs=[
            pl.BlockSpec((1, BN, 1024), lambda i, r: (i, r, 0)),
            pl.BlockSpec((1024, 4096), lambda i, r: (0, 0)),
            pl.BlockSpec((1, 4096), lambda i, r: (0, 0)),
        ],
        out_specs=pl.BlockSpec((1, BN, 4096), lambda i, r: (i, r, 0)),
        out_shape=jax.ShapeDtypeStruct((B, SEQ, 4096), jnp.float32),
    )(h2, w1, b1.reshape(1, 4096))


def _k5b_body(x_ref, g1_ref, w2_ref, b2_ref, xo_ref):
    xo_ref[0] = x_ref[0] + jnp.dot(
        g1_ref[0], w2_ref[...], preferred_element_type=jnp.float32) + b2_ref[0]


def _ffn_down(x, g1, w2, b2):
    B = x.shape[0]
    return pl.pallas_call(
        _k5b_body,
        grid=(B, SEQ // BN),
        in_specs=[
            pl.BlockSpec((1, BN, 1024), lambda i, r: (i, r, 0)),
            pl.BlockSpec((1, BN, 4096), lambda i, r: (i, r, 0)),
            pl.BlockSpec((4096, 1024), lambda i, r: (0, 0)),
            pl.BlockSpec((1, 1024), lambda i, r: (0, 0)),
        ],
        out_specs=pl.BlockSpec((1, BN, 1024), lambda i, r: (i, r, 0)),
        out_shape=jax.ShapeDtypeStruct((B, SEQ, 1024), jnp.float32),
    )(x, g1, w2, b2.reshape(1, 1024))


def _block_tail(x, o, wo, g, b, w1, b1, w2, b2):
    xn, h2 = _oproj_ln(x, o, wo, g, b)
    g1 = _ffn_up(h2, w1, b1)
    return _ffn_down(xn, g1, w2, b2)


def _k6_body(x_ref, w_ref, o_ref):
    xm = jnp.mean(x_ref[0], axis=0, keepdims=True)
    o_ref[0] = jnp.dot(xm, w_ref[...], preferred_element_type=jnp.float32)


def _mean_fc(x, fc_w):
    B = x.shape[0]
    out = pl.pallas_call(
        _k6_body,
        grid=(B,),
        in_specs=[
            pl.BlockSpec((1, SEQ, 1024), lambda i: (i, 0, 0)),
            pl.BlockSpec((1024, 1024), lambda i: (0, 0)),
        ],
        out_specs=pl.BlockSpec((1, 1, 1024), lambda i: (i, 0, 0)),
        out_shape=jax.ShapeDtypeStruct((B, 1, 1024), jnp.float32),
    )(x, fc_w)
    return out.reshape(B, 1024)


_NSLOT = 4
_PIPE = True


def _sc_gather_serial(table, idx2d, chunk):
    info = plsc.get_sparse_core_info()
    nw = info.num_cores * info.num_subcores
    R = idx2d.shape[0] * idx2d.shape[1]
    D = table.shape[1]
    rpw = R // nw
    nch = rpw // chunk
    idx = idx2d.reshape(-1)
    mesh = plsc.VectorSubcoreMesh(core_axis_name="c", subcore_axis_name="s")

    @functools.partial(
        pl.kernel, mesh=mesh,
        out_type=jax.ShapeDtypeStruct((R, D), jnp.float32),
        scratch_types=[
            pltpu.VMEM((chunk,), jnp.int32),
            pltpu.VMEM((chunk, D), jnp.float32),
            pltpu.SemaphoreType.DMA,
        ],
    )
    def k(table_hbm, idx_hbm, out_hbm, idx_v, buf, sem):
        wid = lax.axis_index("s") * info.num_cores + lax.axis_index("c")
        base = wid * rpw

        def body(j, carry):
            off = base + j * chunk
            pltpu.sync_copy(idx_hbm.at[pl.ds(off, chunk)], idx_v)
            pltpu.async_copy(table_hbm.at[idx_v], buf, sem).wait()
            pltpu.sync_copy(buf, out_hbm.at[pl.ds(off, chunk)])
            return carry

        lax.fori_loop(0, nch, body, 0)

    return k(table, idx)


def _sc_scatter_serial(src, idx2d, chunk):
    info = plsc.get_sparse_core_info()
    nw = info.num_cores * info.num_subcores
    R = src.shape[0]
    D = src.shape[1]
    rpw = R // nw
    nch = rpw // chunk
    mesh = plsc.VectorSubcoreMesh(core_axis_name="c", subcore_axis_name="s")

    @functools.partial(
        pl.kernel, mesh=mesh,
        out_type=jax.ShapeDtypeStruct((R, D), jnp.float32),
        scratch_types=[
            pltpu.VMEM((nch, chunk), jnp.int32),
            pltpu.VMEM((chunk, D), jnp.float32),
            pltpu.SemaphoreType.DMA,
        ],
    )
    def k(src_hbm, idx_hbm, out_hbm, idx_v, buf, sem):
        wid = lax.axis_index("s") * info.num_cores + lax.axis_index("c")
        base = wid * rpw
        pltpu.sync_copy(idx_hbm.at[pl.ds(wid * nch, nch)], idx_v)

        def body(j, carry):
            pltpu.sync_copy(src_hbm.at[pl.ds(base + j * chunk, chunk)], buf)
            pltpu.async_copy(buf, out_hbm.at[idx_v.at[j]], sem).wait()
            return carry

        lax.fori_loop(0, nch, body, 0)

    return k(src, idx2d)


def _sc_gather(table, idx2d, chunk):
    info = plsc.get_sparse_core_info()
    nw = info.num_cores * info.num_subcores
    R = idx2d.shape[0] * idx2d.shape[1]
    D = table.shape[1]
    rpw = R // nw
    nch = rpw // chunk
    mesh = plsc.VectorSubcoreMesh(core_axis_name="c", subcore_axis_name="s")

    @functools.partial(
        pl.kernel, mesh=mesh,
        out_type=jax.ShapeDtypeStruct((R, D), jnp.float32),
        scratch_types=[
            pltpu.VMEM((nch, chunk), jnp.int32),
            [pltpu.VMEM((chunk,), jnp.int32) for _ in range(_NSLOT)],
            [pltpu.VMEM((chunk, D), jnp.float32) for _ in range(_NSLOT)],
            [pltpu.SemaphoreType.DMA for _ in range(_NSLOT)],
        ],
    )
    def k(table_hbm, idx_hbm, out_hbm, idx_v, idx1s, bufs, sems):
        wid = lax.axis_index("s") * info.num_cores + lax.axis_index("c")
        base = wid * rpw
        pltpu.sync_copy(idx_hbm.at[pl.ds(wid * nch, nch)], idx_v)

        def start(j, s):
            for i in range(chunk // 16):
                idx1s[s][pl.ds(i * 16, 16)] = idx_v[j, pl.ds(i * 16, 16)]
            pltpu.async_copy(table_hbm.at[idx1s[s]], bufs[s], sems[s])

        for p in range(_NSLOT - 1):
            start(p, p)

        def body(j4, carry):
            for p in range(_NSLOT):
                j = j4 * _NSLOT + p
                pltpu.make_async_copy(
                    table_hbm.at[idx1s[p]], bufs[p], sems[p]).wait()
                pltpu.sync_copy(bufs[p], out_hbm.at[pl.ds(base + j * chunk,
                                                          chunk)])
                nxt = j + _NSLOT - 1
                s = (p + _NSLOT - 1) % _NSLOT

                @pl.when(nxt < nch)
                def _():
                    start(nxt, s)
            return carry

        lax.fori_loop(0, nch // _NSLOT, body, 0)

    return k(table, idx2d)


def _sc_scatter(src, idx2d, chunk):
    info = plsc.get_sparse_core_info()
    nw = info.num_cores * info.num_subcores
    R = src.shape[0]
    D = src.shape[1]
    rpw = R // nw
    nch = rpw // chunk
    mesh = plsc.VectorSubcoreMesh(core_axis_name="c", subcore_axis_name="s")

    @functools.partial(
        pl.kernel, mesh=mesh,
        out_type=jax.ShapeDtypeStruct((R, D), jnp.float32),
        scratch_types=[
            pltpu.VMEM((nch, chunk), jnp.int32),
            [pltpu.VMEM((chunk, D), jnp.float32) for _ in range(_NSLOT)],
            [pltpu.SemaphoreType.DMA for _ in range(_NSLOT)],
        ],
    )
    def k(src_hbm, idx_hbm, out_hbm, idx_v, bufs, sems):
        wid = lax.axis_index("s") * info.num_cores + lax.axis_index("c")
        base = wid * rpw
        pltpu.sync_copy(idx_hbm.at[pl.ds(wid * nch, nch)], idx_v)
        for p in range(_NSLOT - 1):
            pltpu.sync_copy(src_hbm.at[pl.ds(base + p * chunk, chunk)],
                            bufs[p])
            pltpu.async_copy(bufs[p], out_hbm.at[idx_v.at[p]], sems[p])

        def body(j4, carry):
            for p in range(_NSLOT):
                j = j4 * _NSLOT + p
                pltpu.make_async_copy(
                    bufs[p], out_hbm.at[idx_v.at[j]], sems[p]).wait()
                nxt = j + _NSLOT - 1
                s = (p + _NSLOT - 1) % _NSLOT

                @pl.when(nxt < nch)
                def _():
                    pltpu.sync_copy(
                        src_hbm.at[pl.ds(base + nxt * chunk, chunk)], bufs[s])
                    pltpu.async_copy(bufs[s], out_hbm.at[idx_v.at[nxt]],
                                     sems[s])
            return carry

        lax.fori_loop(0, nch // _NSLOT, body, 0)

    return k(src, idx2d)


def _embed(tok_emb, ids):
    return _sc_gather_serial(tok_emb, ids.reshape(-1, 64), 64)


def _sort_rows(qkv_flat, idx):
    f = _sc_scatter if _PIPE else _sc_scatter_serial
    return f(qkv_flat, idx.reshape(-1, 128), 128)


def _unsort_rows(so_flat, idx):
    f = _sc_gather if _PIPE else _sc_gather_serial
    return f(so_flat, idx.reshape(-1, 128), 128)


def kernel(input_ids, tok_emb, pos_emb, Wqk, Wv, Wo, ln1_g, ln1_b, W1, b1,
           W2, b2, ln2_g, ln2_b, fc_W):
    B = input_ids.shape[0]
    slen = input_ids.shape[1]
    if slen < SEQ:
        pad = jnp.zeros((B, SEQ - slen), dtype=input_ids.dtype)
        input_ids = jnp.concatenate([input_ids, pad], axis=1)
    else:
        input_ids = input_ids[:, :SEQ]
    L = Wqk.shape[0]

    ids = input_ids.reshape(-1).astype(jnp.int32)
    x = _embed(tok_emb, ids).reshape(B, SEQ, 1024) + pos_emb[None]

    rkey = jax.random.key(42)
    rots = [
        jax.random.normal(jax.random.fold_in(rkey, i), (DH, NB // 2),
                          dtype=jnp.float32)
        for i in range(L)
    ]

    for i in range(L):
        qkv = _ln_qkv(x, ln1_g[i], ln1_b[i], Wqk[i], Wv[i])
        idx = _lsh_idx(qkv, rots[i]).reshape(-1)
        sqkv = _sort_rows(qkv.reshape(B * SEQ * H, 128), idx)
        so = _attend(sqkv.reshape(B * H, SEQ, 128))
        o = _unsort_rows(so.reshape(B * H * SEQ, 128), idx)
        x = _block_tail(x, o.reshape(B, SEQ, H, 128), Wo[i], ln2_g[i],
                        ln2_b[i], W1[i], b1[i], W2[i], b2[i])

    return _mean_fc(x, fc_W)
